# Initial kernel scaffold; baseline (speedup 1.0000x reference)
import functools

import jax
import jax.numpy as jnp
from jax import lax
from jax.experimental import pallas as pl
from jax.experimental.pallas import tpu as pltpu
from jax.experimental.pallas import tpu_sc as plsc

N = 10000
E = 320000
DIN = 128
H = 8
NG = 16

NE = E + N
NCORE = 2
NSUB = 16
EB = 128
EPT = 20736
NBLK = EPT // EB
E_PAD = EPT * NSUB
NPAD = 10016
ROWS_PT = NPAD // NSUB
HH = H // NCORE


def _sc_edge_kernel(ch):
    F2 = HH * ch
    OUTW = F2 + 16
    NV = F2 // 16

    mesh = plsc.VectorSubcoreMesh(core_axis_name="c", subcore_axis_name="s")

    @functools.partial(
        pl.kernel,
        out_type=jax.ShapeDtypeStruct((NCORE, NPAD, OUTW), jnp.float32),
        mesh=mesh,
        scratch_types=[
            pltpu.VMEM((EB,), jnp.int32),
            pltpu.VMEM((EB,), jnp.int32),
            pltpu.VMEM((EB,), jnp.int32),
            pltpu.VMEM((EB, F2), jnp.float32),
            pltpu.VMEM((EB, F2), jnp.float32),
            pltpu.VMEM((EB, OUTW), jnp.float32),
            pltpu.VMEM((F2,), jnp.float32),
            pltpu.VMEM_SHARED((NPAD, OUTW), jnp.float32),
            pltpu.SemaphoreType.DMA,
            pltpu.SemaphoreType.DMA,
        ],
    )
    def kern(xls, xrs, src_hbm, dst_hbm, att_hbm, out_hbm,
             src_v, dst_v, dst_adj, xl_buf, xr_buf, stage, att_v, acc,
             sem1, sem2):
        c = lax.axis_index("c")
        s = lax.axis_index("s")
        zeros16 = jnp.zeros((16,), jnp.float32)

        def zrow(r, _):
            for i in range(OUTW // 16):
                stage[r, pl.ds(16 * i, 16)] = zeros16
            return 0
        lax.fori_loop(0, EB, zrow, 0)

        r0 = s * ROWS_PT
        for off in range(0, ROWS_PT, EB):
            nrows = min(EB, ROWS_PT - off)
            pltpu.sync_copy(stage.at[pl.ds(0, nrows)],
                            acc.at[pl.ds(r0 + off, nrows)])
        plsc.subcore_barrier()

        pltpu.sync_copy(att_hbm.at[c], att_v)
        attv = [att_v[pl.ds(16 * i, 16)] for i in range(NV)]
        cN = c * N
        lane = lax.iota(jnp.int32, 16)

        def block_body(b, _):
            base = s * EPT + b * EB
            pltpu.sync_copy(src_hbm.at[pl.ds(base, EB)], src_v)
            pltpu.sync_copy(dst_hbm.at[pl.ds(base, EB)], dst_v)
            for i in range(EB // 16):
                sl = pl.ds(16 * i, 16)
                src_v[sl] = src_v[sl] + cN
                dst_adj[sl] = dst_v[sl] + cN
            cp1 = pltpu.async_copy(xls.at[src_v], xl_buf, sem1)
            cp2 = pltpu.async_copy(xrs.at[dst_adj], xr_buf, sem2)
            cp1.wait()
            cp2.wait()

            def edge_body(e):
                xlv = []
                lv = []
                for i in range(NV):
                    sl = pl.ds(16 * i, 16)
                    xv = xl_buf[e, sl]
                    z = xv + xr_buf[e, sl]
                    xlv.append(xv)
                    lv.append(jnp.maximum(z, 0.2 * z))
                ws = []
                whs = []
                if ch >= 16:
                    CV = ch // 16
                    for h in range(HH):
                        t = lv[h * CV] * attv[h * CV]
                        for j in range(1, CV):
                            t = t + lv[h * CV + j] * attv[h * CV + j]
                        sh = jnp.sum(t)
                        wh = jnp.exp(jnp.broadcast_to(sh, (16,)))
                        ws.append(wh)
                        whs.append(wh)
                else:
                    m8 = lane < 8
                    for v in range(NV):
                        t = lv[v] * attv[v]
                        tot = jnp.sum(t)
                        first = jnp.sum(jnp.where(m8, t, 0.0))
                        we = jnp.exp(jnp.broadcast_to(first, (16,)))
                        wo = jnp.exp(jnp.broadcast_to(tot - first, (16,)))
                        ws.append(jnp.where(m8, we, wo))
                        whs.append(we)
                        whs.append(wo)
                if ch >= 16:
                    CV = ch // 16
                    for i in range(NV):
                        stage[e, pl.ds(16 * i, 16)] = xlv[i] * ws[i // CV]
                else:
                    for i in range(NV):
                        stage[e, pl.ds(16 * i, 16)] = xlv[i] * ws[i]
                denv = jnp.zeros((16,), jnp.float32)
                for h in range(HH):
                    denv = jnp.where(lane == h, whs[h], denv)
                stage[e, pl.ds(F2, 16)] = denv

            plsc.parallel_loop(0, EB, 1, unroll=2)(edge_body)

            pltpu.sync_copy(stage, acc.at[dst_v], add=True)
            return 0

        lax.fori_loop(0, NBLK, block_body, 0)

        plsc.subcore_barrier()
        for off in range(0, ROWS_PT, EB):
            nrows = min(EB, ROWS_PT - off)
            pltpu.sync_copy(acc.at[pl.ds(r0 + off, nrows)],
                            out_hbm.at[c, pl.ds(r0 + off, nrows)])

    return kern


def _mm_pair(x, wl, wr, F2):
    K = x.shape[1]
    TR = 2000
    R = N // TR

    def mm_kernel(x_ref, wl_ref, wr_ref, xl_out, xr_out):
        xb = x_ref[...]
        xl_out[0] = jnp.dot(xb, wl_ref[...], preferred_element_type=jnp.float32)
        xr_out[0] = jnp.dot(xb, wr_ref[...], preferred_element_type=jnp.float32)

    out_sh = jax.ShapeDtypeStruct((NCORE, N, F2), jnp.float32)
    xl3, xr3 = pl.pallas_call(
        mm_kernel,
        grid=(NCORE, R),
        in_specs=[
            pl.BlockSpec((TR, K), lambda c, r: (r, 0)),
            pl.BlockSpec((K, F2), lambda c, r: (0, c)),
            pl.BlockSpec((K, F2), lambda c, r: (0, c)),
        ],
        out_specs=[
            pl.BlockSpec((1, TR, F2), lambda c, r: (c, r, 0)),
            pl.BlockSpec((1, TR, F2), lambda c, r: (c, r, 0)),
        ],
        out_shape=[out_sh, out_sh],
    )(x, wl, wr)
    return xl3.reshape(NCORE * N, F2), xr3.reshape(NCORE * N, F2)


def _combine(acc, gw, gb, gm, bias, ch):
    F2 = HH * ch
    F = H * ch

    def comb_kernel(acc_ref, gw_ref, gb_ref, gm_ref, b_ref, out_ref):
        parts = []
        for c in range(NCORE):
            num = acc_ref[c, :N, :F2]
            for h in range(HH):
                den = acc_ref[c, :N, F2 + h:F2 + h + 1]
                parts.append(num[:, h * ch:(h + 1) * ch] / (den + 1e-16))
        out0 = jnp.concatenate(parts, axis=1) + b_ref[...]
        mean = jnp.mean(out0, axis=0, keepdims=True)
        cent = out0 - gm_ref[...] * mean
        var = jnp.mean(cent * cent, axis=0, keepdims=True)
        y = gw_ref[...] * cent / jnp.sqrt(var + 1e-5) + gb_ref[...]
        out_ref[...] = jnp.maximum(y, 0.0)

    return pl.pallas_call(
        comb_kernel,
        out_shape=jax.ShapeDtypeStruct((N, F), jnp.float32),
    )(acc, gw.reshape(1, F), gb.reshape(1, F), gm.reshape(1, F),
      bias.reshape(1, F))


def _finalize(acc, batch, gw, gb, gm, b3, wlin, blin):
    ch = 8
    F2 = HH * ch

    def fin_kernel(acc_ref, batch_ref, gw_ref, gb_ref, gm_ref, b_ref,
                   wlin_ref, blin_ref, logits_ref, pooled_ref):
        segs = []
        for c in range(NCORE):
            num = acc_ref[c, :N, :F2]
            for h in range(HH):
                den = acc_ref[c, :N, F2 + h:F2 + h + 1]
                segs.append(num[:, h * ch:(h + 1) * ch] / (den + 1e-16))
        hsum = segs[0]
        for sg in segs[1:]:
            hsum = hsum + sg
        out0 = hsum / float(H) + b_ref[...]
        mean = jnp.mean(out0, axis=0, keepdims=True)
        cent = out0 - gm_ref[...] * mean
        var = jnp.mean(cent * cent, axis=0, keepdims=True)
        h3 = jnp.maximum(gw_ref[...] * cent / jnp.sqrt(var + 1e-5)
                         + gb_ref[...], 0.0)
        gid = lax.broadcasted_iota(jnp.int32, (1, NG), 1)
        oh = (batch_ref[...] == gid).astype(jnp.float32)
        psum = lax.dot_general(oh, h3, (((0,), (0,)), ((), ())),
                               preferred_element_type=jnp.float32)
        counts = jnp.sum(oh, axis=0, keepdims=True)
        pooled = psum / jnp.maximum(counts, 1.0).reshape(NG, 1)
        pooled_ref[...] = pooled
        logits_ref[...] = jnp.dot(pooled, wlin_ref[...],
                                  preferred_element_type=jnp.float32) \
            + blin_ref[...]

    return pl.pallas_call(
        fin_kernel,
        out_shape=[jax.ShapeDtypeStruct((NG, 4), jnp.float32),
                   jax.ShapeDtypeStruct((NG, 8), jnp.float32)],
    )(acc, batch.reshape(N, 1), gw.reshape(1, 8), gb.reshape(1, 8),
      gm.reshape(1, 8), b3.reshape(1, 8), wlin, blin.reshape(1, 4))


_sc_l1 = _sc_edge_kernel(32)
_sc_l2 = _sc_edge_kernel(16)
_sc_l3 = _sc_edge_kernel(8)


def kernel(x, edge_index, batch, W1l, W1r, a1, b1, gn1w, gn1b, gn1m,
           W2l, W2r, a2, b2, gn2w, gn2b, gn2m,
           W3l, W3r, a3, b3, gn3w, gn3b, gn3m, Wlin, blin):
    src = jnp.concatenate([
        edge_index[0].astype(jnp.int32),
        jnp.arange(N, dtype=jnp.int32),
        jnp.zeros((E_PAD - NE,), jnp.int32),
    ])
    dst = jnp.concatenate([
        edge_index[1].astype(jnp.int32),
        jnp.arange(N, dtype=jnp.int32),
        jnp.full((E_PAD - NE,), N, jnp.int32),
    ])
    batch32 = batch.astype(jnp.int32)

    h = x
    for (Wl, Wr, att, bias, gwn, gbn, gmn, ch) in (
        (W1l, W1r, a1, b1, gn1w, gn1b, gn1m, 32),
        (W2l, W2r, a2, b2, gn2w, gn2b, gn2m, 16),
    ):
        F2 = HH * ch
        xls, xrs = _mm_pair(h, Wl, Wr, F2)
        att2 = att.reshape(NCORE, F2)
        acc = (_sc_l1 if ch == 32 else _sc_l2)(xls, xrs, src, dst, att2)
        h = _combine(acc, gwn, gbn, gmn, bias, ch)

    xls, xrs = _mm_pair(h, W3l, W3r, HH * 8)
    att2 = a3.reshape(NCORE, HH * 8)
    acc = _sc_l3(xls, xrs, src, dst, att2)
    return _finalize(acc, batch32, gn3w, gn3b, gn3m, b3, Wlin, blin)


# speedup vs baseline: 53.8291x; 53.8291x over previous
import functools

import jax
import jax.numpy as jnp
from jax import lax
from jax.experimental import pallas as pl
from jax.experimental.pallas import tpu as pltpu
from jax.experimental.pallas import tpu_sc as plsc

N = 10000
E = 320000
DIN = 128
H = 8
NG = 16

NE = E + N
NCORE = 2
NSUB = 16
EPT = 20736
E_PAD = EPT * NSUB
NPAD = 10112
ROWS_PT = NPAD // NSUB
HH = H // NCORE


def _sc_edge_kernel(ch, EB):
    F2 = HH * ch
    OUTW = F2 + 16
    NV = F2 // 16
    NBLK = EPT // EB

    mesh = plsc.VectorSubcoreMesh(core_axis_name="c", subcore_axis_name="s")

    @functools.partial(
        pl.kernel,
        out_type=jax.ShapeDtypeStruct((NCORE, NPAD, OUTW), jnp.float32),
        mesh=mesh,
        scratch_types=[
            pltpu.VMEM((EB,), jnp.int32),
            pltpu.VMEM((EB,), jnp.int32),
            pltpu.VMEM((EB,), jnp.int32),
            pltpu.VMEM((EB, F2), jnp.float32),
            pltpu.VMEM((EB, F2), jnp.float32),
            pltpu.VMEM((EB, OUTW), jnp.float32),
            pltpu.VMEM((F2,), jnp.float32),
            pltpu.VMEM_SHARED((NPAD, OUTW), jnp.float32),
            pltpu.SemaphoreType.DMA,
            pltpu.SemaphoreType.DMA,
        ],
        compiler_params=pltpu.CompilerParams(use_tc_tiling_on_sc=False,
                                             needs_layout_passes=False),
    )
    def kern(xls, xrs, src_hbm, dst_hbm, att_hbm, out_hbm,
             src_v, dst_v, dst_adj, xl_buf, xr_buf, stage, att_v, acc,
             sem1, sem2):
        c = lax.axis_index("c")
        s = lax.axis_index("s")
        zeros16 = jnp.zeros((16,), jnp.float32)

        def zrow(r, _):
            for i in range(OUTW // 16):
                stage[r, pl.ds(16 * i, 16)] = zeros16
            return 0
        lax.fori_loop(0, EB, zrow, 0)

        r0 = s * ROWS_PT
        for off in range(0, ROWS_PT, EB):
            nrows = min(EB, ROWS_PT - off)
            pltpu.sync_copy(stage.at[pl.ds(0, nrows)],
                            acc.at[pl.ds(r0 + off, nrows)])
        plsc.subcore_barrier()

        pltpu.sync_copy(att_hbm.at[c], att_v)
        attv = [att_v[pl.ds(16 * i, 16)] for i in range(NV)]
        cN = c * N
        lane = lax.iota(jnp.int32, 16)

        def block_body(b, _):
            base = s * EPT + b * EB
            pltpu.sync_copy(src_hbm.at[pl.ds(base, EB)], src_v)
            pltpu.sync_copy(dst_hbm.at[pl.ds(base, EB)], dst_v)
            for i in range(EB // 16):
                sl = pl.ds(16 * i, 16)
                src_v[sl] = src_v[sl] + cN
                dst_adj[sl] = dst_v[sl] + cN
            cp1 = pltpu.async_copy(xls.at[src_v], xl_buf, sem1)
            cp2 = pltpu.async_copy(xrs.at[dst_adj], xr_buf, sem2)
            cp1.wait()
            cp2.wait()

            def edge_body(e):
                xlv = []
                lv = []
                for i in range(NV):
                    sl = pl.ds(16 * i, 16)
                    xv = xl_buf[e, sl]
                    z = xv + xr_buf[e, sl]
                    xlv.append(xv)
                    lv.append(jnp.maximum(z, 0.2 * z))
                ws = []
                whs = []
                if ch >= 16:
                    CV = ch // 16
                    for h in range(HH):
                        t = lv[h * CV] * attv[h * CV]
                        for j in range(1, CV):
                            t = t + lv[h * CV + j] * attv[h * CV + j]
                        sh = jnp.sum(t)
                        wh = jnp.exp(jnp.broadcast_to(sh, (16,)))
                        ws.append(wh)
                        whs.append(wh)
                else:
                    m8 = lane < 8
                    for v in range(NV):
                        t = lv[v] * attv[v]
                        tot = jnp.sum(t)
                        first = jnp.sum(jnp.where(m8, t, 0.0))
                        we = jnp.exp(jnp.broadcast_to(first, (16,)))
                        wo = jnp.exp(jnp.broadcast_to(tot - first, (16,)))
                        ws.append(jnp.where(m8, we, wo))
                        whs.append(we)
                        whs.append(wo)
                if ch >= 16:
                    CV = ch // 16
                    for i in range(NV):
                        stage[e, pl.ds(16 * i, 16)] = xlv[i] * ws[i // CV]
                else:
                    for i in range(NV):
                        stage[e, pl.ds(16 * i, 16)] = xlv[i] * ws[i]
                denv = jnp.zeros((16,), jnp.float32)
                for h in range(HH):
                    denv = jnp.where(lane == h, whs[h], denv)
                stage[e, pl.ds(F2, 16)] = denv

            plsc.parallel_loop(0, EB, 1, unroll=2)(edge_body)

            pltpu.sync_copy(stage, acc.at[dst_v], add=True)
            return 0

        lax.fori_loop(0, NBLK, block_body, 0)

        plsc.subcore_barrier()
        for off in range(0, ROWS_PT, EB):
            nrows = min(EB, ROWS_PT - off)
            pltpu.sync_copy(acc.at[pl.ds(r0 + off, nrows)],
                            out_hbm.at[c, pl.ds(r0 + off, nrows)])

    return kern


def _mm_pair(x, wl, wr, F2):
    K = x.shape[1]
    TR = 2000
    R = N // TR
    wlh = wl.reshape(K, NCORE, F2).transpose(1, 0, 2)
    wrh = wr.reshape(K, NCORE, F2).transpose(1, 0, 2)

    def mm_kernel(x_ref, wl_ref, wr_ref, xl_out, xr_out):
        xb = x_ref[...]
        xl_out[0] = jnp.dot(xb, wl_ref[0], preferred_element_type=jnp.float32)
        xr_out[0] = jnp.dot(xb, wr_ref[0], preferred_element_type=jnp.float32)

    out_sh = jax.ShapeDtypeStruct((NCORE, N, F2), jnp.float32)
    xl3, xr3 = pl.pallas_call(
        mm_kernel,
        grid=(NCORE, R),
        in_specs=[
            pl.BlockSpec((TR, K), lambda c, r: (r, 0)),
            pl.BlockSpec((1, K, F2), lambda c, r: (c, 0, 0)),
            pl.BlockSpec((1, K, F2), lambda c, r: (c, 0, 0)),
        ],
        out_specs=[
            pl.BlockSpec((1, TR, F2), lambda c, r: (c, r, 0)),
            pl.BlockSpec((1, TR, F2), lambda c, r: (c, r, 0)),
        ],
        out_shape=[out_sh, out_sh],
    )(x, wlh, wrh)
    return xl3.reshape(NCORE * N, F2), xr3.reshape(NCORE * N, F2)


TRW = 1264
RSTEPS = NPAD // TRW


def _div_stats(acc, bias, ch, mean_heads):
    F2 = HH * ch
    F = H * ch
    Fout = ch if mean_heads else F
    OUTW = F2 + 16

    def k(acc_ref, b_ref, out_ref, s1_ref, s2_ref):
        r = pl.program_id(0)
        parts = []
        for c in range(NCORE):
            num = acc_ref[c, :, :F2]
            for h in range(HH):
                den = acc_ref[c, :, F2 + h:F2 + h + 1]
                parts.append(num[:, h * ch:(h + 1) * ch] / (den + 1e-16))
        if mean_heads:
            t = parts[0]
            for p in parts[1:]:
                t = t + p
            out0 = t / float(H) + b_ref[...]
        else:
            out0 = jnp.concatenate(parts, axis=1) + b_ref[...]
        out_ref[...] = out0
        row = r * TRW + lax.broadcasted_iota(jnp.int32, (TRW, 1), 0)
        mask = row < N
        c1 = jnp.sum(jnp.where(mask, out0, 0.0), axis=0, keepdims=True)
        c2 = jnp.sum(jnp.where(mask, out0 * out0, 0.0), axis=0,
                     keepdims=True)

        @pl.when(r == 0)
        def _():
            s1_ref[...] = c1
            s2_ref[...] = c2

        @pl.when(r > 0)
        def _():
            s1_ref[...] += c1
            s2_ref[...] += c2

    stat_sh = jax.ShapeDtypeStruct((1, Fout), jnp.float32)
    return pl.pallas_call(
        k,
        grid=(RSTEPS,),
        in_specs=[
            pl.BlockSpec((NCORE, TRW, OUTW), lambda r: (0, r, 0)),
            pl.BlockSpec((1, Fout), lambda r: (0, 0)),
        ],
        out_specs=[
            pl.BlockSpec((TRW, Fout), lambda r: (r, 0)),
            pl.BlockSpec((1, Fout), lambda r: (0, 0)),
            pl.BlockSpec((1, Fout), lambda r: (0, 0)),
        ],
        out_shape=[jax.ShapeDtypeStruct((N, Fout), jnp.float32),
                   stat_sh, stat_sh],
    )(acc, bias.reshape(1, Fout))


def _norm_relu(out0, s1, s2, gw, gb, gm):
    F = out0.shape[1]

    def k(x_ref, s1_ref, s2_ref, gw_ref, gb_ref, gm_ref, out_ref):
        mean = s1_ref[...] * (1.0 / N)
        ex2 = s2_ref[...] * (1.0 / N)
        gm_ = gm_ref[...]
        var = ex2 + (gm_ * gm_ - 2.0 * gm_) * mean * mean
        inv = gw_ref[...] / jnp.sqrt(var + 1e-5)
        y = (x_ref[...] - gm_ * mean) * inv + gb_ref[...]
        out_ref[...] = jnp.maximum(y, 0.0)

    vec = pl.BlockSpec((1, F), lambda r: (0, 0))
    return pl.pallas_call(
        k,
        grid=(RSTEPS,),
        in_specs=[pl.BlockSpec((TRW, F), lambda r: (r, 0)),
                  vec, vec, vec, vec, vec],
        out_specs=pl.BlockSpec((TRW, F), lambda r: (r, 0)),
        out_shape=jax.ShapeDtypeStruct((N, F), jnp.float32),
    )(out0, s1, s2, gw.reshape(1, F), gb.reshape(1, F), gm.reshape(1, F))


def _pool(out0, s1, s2, batch, gw, gb, gm, wlin, blin):
    def k(x_ref, s1_ref, s2_ref, batch_ref, gw_ref, gb_ref, gm_ref,
          wlin_ref, blin_ref, logits_ref, pooled_ref):
        mean = s1_ref[...] * (1.0 / N)
        ex2 = s2_ref[...] * (1.0 / N)
        gm_ = gm_ref[...]
        var = ex2 + (gm_ * gm_ - 2.0 * gm_) * mean * mean
        inv = gw_ref[...] / jnp.sqrt(var + 1e-5)
        h3 = jnp.maximum((x_ref[...] - gm_ * mean) * inv + gb_ref[...], 0.0)
        gid = lax.broadcasted_iota(jnp.int32, (1, NG), 1)
        oh = (batch_ref[...] == gid).astype(jnp.float32)
        psum = lax.dot_general(oh, h3, (((0,), (0,)), ((), ())),
                               preferred_element_type=jnp.float32)
        counts = jnp.sum(oh, axis=0, keepdims=True)
        pooled = psum / jnp.maximum(counts, 1.0).reshape(NG, 1)
        pooled_ref[...] = pooled
        logits_ref[...] = jnp.dot(pooled, wlin_ref[...],
                                  preferred_element_type=jnp.float32) \
            + blin_ref[...]

    return pl.pallas_call(
        k,
        out_shape=[jax.ShapeDtypeStruct((NG, 4), jnp.float32),
                   jax.ShapeDtypeStruct((NG, 8), jnp.float32)],
    )(out0, s1, s2, batch.reshape(N, 1), gw.reshape(1, 8), gb.reshape(1, 8),
      gm.reshape(1, 8), wlin, blin.reshape(1, 4))


_sc_l1 = _sc_edge_kernel(32, 64)
_sc_l2 = _sc_edge_kernel(16, 128)
_sc_l3 = _sc_edge_kernel(8, 128)


def kernel(x, edge_index, batch, W1l, W1r, a1, b1, gn1w, gn1b, gn1m,
           W2l, W2r, a2, b2, gn2w, gn2b, gn2m,
           W3l, W3r, a3, b3, gn3w, gn3b, gn3m, Wlin, blin):
    src = jnp.concatenate([
        edge_index[0].astype(jnp.int32),
        jnp.arange(N, dtype=jnp.int32),
        jnp.zeros((E_PAD - NE,), jnp.int32),
    ])
    dst = jnp.concatenate([
        edge_index[1].astype(jnp.int32),
        jnp.arange(N, dtype=jnp.int32),
        jnp.full((E_PAD - NE,), N, jnp.int32),
    ])
    batch32 = batch.astype(jnp.int32)

    h = x
    for (Wl, Wr, att, bias, gwn, gbn, gmn, ch) in (
        (W1l, W1r, a1, b1, gn1w, gn1b, gn1m, 32),
        (W2l, W2r, a2, b2, gn2w, gn2b, gn2m, 16),
    ):
        F2 = HH * ch
        xls, xrs = _mm_pair(h, Wl, Wr, F2)
        att2 = att.reshape(NCORE, F2)
        acc = (_sc_l1 if ch == 32 else _sc_l2)(xls, xrs, src, dst, att2)
        out0, s1, s2 = _div_stats(acc, bias, ch, mean_heads=False)
        h = _norm_relu(out0, s1, s2, gwn, gbn, gmn)

    xls, xrs = _mm_pair(h, W3l, W3r, HH * 8)
    att2 = a3.reshape(NCORE, HH * 8)
    acc = _sc_l3(xls, xrs, src, dst, att2)
    out0, s1, s2 = _div_stats(acc, b3, 8, mean_heads=True)
    return _pool(out0, s1, s2, batch32, gn3w, gn3b, gn3m, Wlin, blin)


# speedup vs baseline: 105.2864x; 1.9559x over previous
import functools

import jax
import jax.numpy as jnp
from jax import lax
from jax.experimental import pallas as pl
from jax.experimental.pallas import tpu as pltpu
from jax.experimental.pallas import tpu_sc as plsc

N = 10000
E = 320000
DIN = 128
H = 8
NG = 16

NE = E + N
NCORE = 2
NSUB = 16
EPT = 20736
E_PAD = EPT * NSUB
NPAD = 10112
ROWS_PT = NPAD // NSUB
HH = H // NCORE


def _sc_edge_kernel(ch, EB, unroll=2):
    F2 = HH * ch
    OUTW = F2 + 16
    NV = F2 // 16
    NBLK = EPT // EB

    mesh = plsc.VectorSubcoreMesh(core_axis_name="c", subcore_axis_name="s")

    idx_t = pltpu.VMEM((EB,), jnp.int32)
    row_t = pltpu.VMEM((EB, F2), jnp.float32)
    stage_t = pltpu.VMEM((EB, OUTW), jnp.float32)

    @functools.partial(
        pl.kernel,
        out_type=jax.ShapeDtypeStruct((NCORE, NPAD, OUTW), jnp.float32),
        mesh=mesh,
        scratch_types=(
            [idx_t] * 8
            + [row_t] * 4
            + [stage_t] * 2
            + [pltpu.VMEM((F2,), jnp.float32)]
            + [pltpu.VMEM_SHARED((NPAD, OUTW), jnp.float32)]
            + [pltpu.SemaphoreType.DMA] * 8
        ),
        compiler_params=pltpu.CompilerParams(use_tc_tiling_on_sc=False,
                                             needs_layout_passes=False),
    )
    def kern(xls, xrs, sadj_hbm, dadj_hbm, draw_hbm, att_hbm, out_hbm,
             sadjA, dadjA, drawA, dscatA, sadjB, dadjB, drawB, dscatB,
             xlA, xrA, xlB, xrB, stageA, stageB, att_v, acc,
             sem_iA, sem_iB, sem_xlA, sem_xlB, sem_xrA, sem_xrB,
             sem_scA, sem_scB):
        c = lax.axis_index("c")
        s = lax.axis_index("s")
        zeros16 = jnp.zeros((16,), jnp.float32)

        A = (sadjA, dadjA, drawA, dscatA, xlA, xrA, stageA,
             sem_iA, sem_xlA, sem_xrA, sem_scA)
        B = (sadjB, dadjB, drawB, dscatB, xlB, xrB, stageB,
             sem_iB, sem_xlB, sem_xrB, sem_scB)

        def zrow(r, _):
            for i in range(OUTW // 16):
                stageA[r, pl.ds(16 * i, 16)] = zeros16
            return 0
        lax.fori_loop(0, EB, zrow, 0)

        r0 = s * ROWS_PT
        for off in range(0, ROWS_PT, EB):
            nrows = min(EB, ROWS_PT - off)
            pltpu.sync_copy(stageA.at[pl.ds(0, nrows)],
                            acc.at[pl.ds(r0 + off, nrows)])
        plsc.subcore_barrier()

        pltpu.sync_copy(att_hbm.at[c], att_v)
        attv = [att_v[pl.ds(16 * i, 16)] for i in range(NV)]
        lane = lax.iota(jnp.int32, 16)
        ebase = s * EPT

        def idx_srcs(blk):
            base = ebase + blk * EB
            return (sadj_hbm.at[c, pl.ds(base, EB)],
                    dadj_hbm.at[c, pl.ds(base, EB)],
                    draw_hbm.at[pl.ds(base, EB)])

        def issue_idx(st, blk):
            sa, da, dr = idx_srcs(blk)
            pltpu.async_copy(sa, st[0], st[7])
            pltpu.async_copy(da, st[1], st[7])
            pltpu.async_copy(dr, st[2], st[7])

        def wait_idx(st, blk):
            sa, da, dr = idx_srcs(blk)
            pltpu.make_async_copy(sa, st[0], st[7]).wait()
            pltpu.make_async_copy(da, st[1], st[7]).wait()
            pltpu.make_async_copy(dr, st[2], st[7]).wait()

        def issue_gathers(st):
            pltpu.async_copy(xls.at[st[0]], st[4], st[8])
            pltpu.async_copy(xrs.at[st[1]], st[5], st[9])

        def wait_gathers(st):
            pltpu.make_async_copy(xls.at[st[0]], st[4], st[8]).wait()
            pltpu.make_async_copy(xrs.at[st[1]], st[5], st[9]).wait()

        def issue_scatter(st):
            pltpu.async_copy(st[6], acc.at[st[3]], st[10], add=True)

        def wait_scatter(st):
            pltpu.make_async_copy(st[6], acc.at[st[3]], st[10]).wait()

        def snap_scatter_idx(st):
            for i in range(EB // 16):
                sl = pl.ds(16 * i, 16)
                st[3][sl] = st[2][sl]

        def compute(st):
            xl_buf, xr_buf, stage = st[4], st[5], st[6]

            def edge_body(e):
                xlv = []
                lv = []
                for i in range(NV):
                    sl = pl.ds(16 * i, 16)
                    xv = xl_buf[e, sl]
                    z = xv + xr_buf[e, sl]
                    xlv.append(xv)
                    lv.append(jnp.maximum(z, 0.2 * z))
                ws = []
                whs = []
                if ch >= 16:
                    CV = ch // 16
                    for h in range(HH):
                        t = lv[h * CV] * attv[h * CV]
                        for j in range(1, CV):
                            t = t + lv[h * CV + j] * attv[h * CV + j]
                        sh = jnp.sum(t)
                        wh = jnp.exp(jnp.broadcast_to(sh, (16,)))
                        ws.append(wh)
                        whs.append(wh)
                else:
                    m8 = lane < 8
                    for v in range(NV):
                        t = lv[v] * attv[v]
                        tot = jnp.sum(t)
                        first = jnp.sum(jnp.where(m8, t, 0.0))
                        we = jnp.exp(jnp.broadcast_to(first, (16,)))
                        wo = jnp.exp(jnp.broadcast_to(tot - first, (16,)))
                        ws.append(jnp.where(m8, we, wo))
                        whs.append(we)
                        whs.append(wo)
                if ch >= 16:
                    CV = ch // 16
                    for i in range(NV):
                        stage[e, pl.ds(16 * i, 16)] = xlv[i] * ws[i // CV]
                else:
                    for i in range(NV):
                        stage[e, pl.ds(16 * i, 16)] = xlv[i] * ws[i]
                denv = jnp.zeros((16,), jnp.float32)
                for h in range(HH):
                    denv = jnp.where(lane == h, whs[h], denv)
                stage[e, pl.ds(F2, 16)] = denv

            plsc.parallel_loop(0, EB, 1, unroll=unroll)(edge_body)

        for ref, src in zip((sadjA, dadjA, drawA), idx_srcs(0)):
            pltpu.sync_copy(src, ref)
        issue_gathers(A)
        issue_idx(B, 1)

        def pair_body(b2, _):
            blk0 = 2 * b2
            wait_gathers(A)
            wait_idx(B, blk0 + 1)
            issue_gathers(B)

            @pl.when(b2 > 0)
            def _():
                wait_scatter(A)
            snap_scatter_idx(A)

            @pl.when(blk0 + 2 < NBLK)
            def _():
                issue_idx(A, blk0 + 2)
            compute(A)
            issue_scatter(A)

            wait_gathers(B)

            @pl.when(blk0 + 2 < NBLK)
            def _():
                wait_idx(A, blk0 + 2)
                issue_gathers(A)

            @pl.when(b2 > 0)
            def _():
                wait_scatter(B)
            snap_scatter_idx(B)

            @pl.when(blk0 + 3 < NBLK)
            def _():
                issue_idx(B, blk0 + 3)
            compute(B)
            issue_scatter(B)
            return 0

        lax.fori_loop(0, NBLK // 2, pair_body, 0)
        wait_scatter(A)
        wait_scatter(B)

        plsc.subcore_barrier()
        for off in range(0, ROWS_PT, EB):
            nrows = min(EB, ROWS_PT - off)
            pltpu.sync_copy(acc.at[pl.ds(r0 + off, nrows)],
                            out_hbm.at[c, pl.ds(r0 + off, nrows)])

    return kern


def _mm_pair(x, wl, wr, F2):
    K = x.shape[1]
    TR = 2000
    R = N // TR
    wlh = wl.reshape(K, NCORE, F2).transpose(1, 0, 2)
    wrh = wr.reshape(K, NCORE, F2).transpose(1, 0, 2)

    def mm_kernel(x_ref, wl_ref, wr_ref, xl_out, xr_out):
        xb = x_ref[...]
        xl_out[0] = jnp.dot(xb, wl_ref[0], preferred_element_type=jnp.float32)
        xr_out[0] = jnp.dot(xb, wr_ref[0], preferred_element_type=jnp.float32)

    out_sh = jax.ShapeDtypeStruct((NCORE, N, F2), jnp.float32)
    xl3, xr3 = pl.pallas_call(
        mm_kernel,
        grid=(NCORE, R),
        in_specs=[
            pl.BlockSpec((TR, K), lambda c, r: (r, 0)),
            pl.BlockSpec((1, K, F2), lambda c, r: (c, 0, 0)),
            pl.BlockSpec((1, K, F2), lambda c, r: (c, 0, 0)),
        ],
        out_specs=[
            pl.BlockSpec((1, TR, F2), lambda c, r: (c, r, 0)),
            pl.BlockSpec((1, TR, F2), lambda c, r: (c, r, 0)),
        ],
        out_shape=[out_sh, out_sh],
    )(x, wlh, wrh)
    return xl3.reshape(NCORE * N, F2), xr3.reshape(NCORE * N, F2)


TRW = 1264
RSTEPS = NPAD // TRW


def _div_stats(acc, bias, ch, mean_heads):
    F2 = HH * ch
    F = H * ch
    Fout = ch if mean_heads else F
    OUTW = F2 + 16

    def k(acc_ref, b_ref, out_ref, s1_ref, s2_ref):
        r = pl.program_id(0)
        parts = []
        for c in range(NCORE):
            num = acc_ref[c, :, :F2]
            for h in range(HH):
                den = acc_ref[c, :, F2 + h:F2 + h + 1]
                parts.append(num[:, h * ch:(h + 1) * ch] / (den + 1e-16))
        if mean_heads:
            t = parts[0]
            for p in parts[1:]:
                t = t + p
            out0 = t / float(H) + b_ref[...]
        else:
            out0 = jnp.concatenate(parts, axis=1) + b_ref[...]
        out_ref[...] = out0
        row = r * TRW + lax.broadcasted_iota(jnp.int32, (TRW, 1), 0)
        mask = row < N
        c1 = jnp.sum(jnp.where(mask, out0, 0.0), axis=0, keepdims=True)
        c2 = jnp.sum(jnp.where(mask, out0 * out0, 0.0), axis=0,
                     keepdims=True)

        @pl.when(r == 0)
        def _():
            s1_ref[...] = c1
            s2_ref[...] = c2

        @pl.when(r > 0)
        def _():
            s1_ref[...] += c1
            s2_ref[...] += c2

    stat_sh = jax.ShapeDtypeStruct((1, Fout), jnp.float32)
    return pl.pallas_call(
        k,
        grid=(RSTEPS,),
        in_specs=[
            pl.BlockSpec((NCORE, TRW, OUTW), lambda r: (0, r, 0)),
            pl.BlockSpec((1, Fout), lambda r: (0, 0)),
        ],
        out_specs=[
            pl.BlockSpec((TRW, Fout), lambda r: (r, 0)),
            pl.BlockSpec((1, Fout), lambda r: (0, 0)),
            pl.BlockSpec((1, Fout), lambda r: (0, 0)),
        ],
        out_shape=[jax.ShapeDtypeStruct((N, Fout), jnp.float32),
                   stat_sh, stat_sh],
    )(acc, bias.reshape(1, Fout))


def _norm_relu(out0, s1, s2, gw, gb, gm):
    F = out0.shape[1]

    def k(x_ref, s1_ref, s2_ref, gw_ref, gb_ref, gm_ref, out_ref):
        mean = s1_ref[...] * (1.0 / N)
        ex2 = s2_ref[...] * (1.0 / N)
        gm_ = gm_ref[...]
        var = ex2 + (gm_ * gm_ - 2.0 * gm_) * mean * mean
        inv = gw_ref[...] / jnp.sqrt(var + 1e-5)
        y = (x_ref[...] - gm_ * mean) * inv + gb_ref[...]
        out_ref[...] = jnp.maximum(y, 0.0)

    vec = pl.BlockSpec((1, F), lambda r: (0, 0))
    return pl.pallas_call(
        k,
        grid=(RSTEPS,),
        in_specs=[pl.BlockSpec((TRW, F), lambda r: (r, 0)),
                  vec, vec, vec, vec, vec],
        out_specs=pl.BlockSpec((TRW, F), lambda r: (r, 0)),
        out_shape=jax.ShapeDtypeStruct((N, F), jnp.float32),
    )(out0, s1, s2, gw.reshape(1, F), gb.reshape(1, F), gm.reshape(1, F))


def _pool(out0, s1, s2, batch, gw, gb, gm, wlin, blin):
    def k(x_ref, s1_ref, s2_ref, batch_ref, gw_ref, gb_ref, gm_ref,
          wlin_ref, blin_ref, logits_ref, pooled_ref):
        mean = s1_ref[...] * (1.0 / N)
        ex2 = s2_ref[...] * (1.0 / N)
        gm_ = gm_ref[...]
        var = ex2 + (gm_ * gm_ - 2.0 * gm_) * mean * mean
        inv = gw_ref[...] / jnp.sqrt(var + 1e-5)
        h3 = jnp.maximum((x_ref[...] - gm_ * mean) * inv + gb_ref[...], 0.0)
        gid = lax.broadcasted_iota(jnp.int32, (1, NG), 1)
        oh = (batch_ref[...] == gid).astype(jnp.float32)
        psum = lax.dot_general(oh, h3, (((0,), (0,)), ((), ())),
                               preferred_element_type=jnp.float32)
        counts = jnp.sum(oh, axis=0, keepdims=True)
        pooled = psum / jnp.maximum(counts, 1.0).reshape(NG, 1)
        pooled_ref[...] = pooled
        logits_ref[...] = jnp.dot(pooled, wlin_ref[...],
                                  preferred_element_type=jnp.float32) \
            + blin_ref[...]

    return pl.pallas_call(
        k,
        out_shape=[jax.ShapeDtypeStruct((NG, 4), jnp.float32),
                   jax.ShapeDtypeStruct((NG, 8), jnp.float32)],
    )(out0, s1, s2, batch.reshape(N, 1), gw.reshape(1, 8), gb.reshape(1, 8),
      gm.reshape(1, 8), wlin, blin.reshape(1, 4))


_sc_l1 = _sc_edge_kernel(32, 48)
_sc_l2 = _sc_edge_kernel(16, 128)
_sc_l3 = _sc_edge_kernel(8, 128)


def kernel(x, edge_index, batch, W1l, W1r, a1, b1, gn1w, gn1b, gn1m,
           W2l, W2r, a2, b2, gn2w, gn2b, gn2m,
           W3l, W3r, a3, b3, gn3w, gn3b, gn3m, Wlin, blin):
    src = jnp.concatenate([
        edge_index[0].astype(jnp.int32),
        jnp.arange(N, dtype=jnp.int32),
        jnp.zeros((E_PAD - NE,), jnp.int32),
    ])
    dst = jnp.concatenate([
        edge_index[1].astype(jnp.int32),
        jnp.arange(N, dtype=jnp.int32),
        jnp.full((E_PAD - NE,), N, jnp.int32),
    ])
    coff = jnp.array([[0], [N]], jnp.int32)
    sadj = src[None, :] + coff
    dadj = dst[None, :] + coff
    batch32 = batch.astype(jnp.int32)

    h = x
    for (Wl, Wr, att, bias, gwn, gbn, gmn, ch) in (
        (W1l, W1r, a1, b1, gn1w, gn1b, gn1m, 32),
        (W2l, W2r, a2, b2, gn2w, gn2b, gn2m, 16),
    ):
        F2 = HH * ch
        xls, xrs = _mm_pair(h, Wl, Wr, F2)
        att2 = att.reshape(NCORE, F2)
        acc = (_sc_l1 if ch == 32 else _sc_l2)(xls, xrs, sadj, dadj, dst,
                                               att2)
        out0, s1, s2 = _div_stats(acc, bias, ch, mean_heads=False)
        h = _norm_relu(out0, s1, s2, gwn, gbn, gmn)

    xls, xrs = _mm_pair(h, W3l, W3r, HH * 8)
    att2 = a3.reshape(NCORE, HH * 8)
    acc = _sc_l3(xls, xrs, sadj, dadj, dst, att2)
    out0, s1, s2 = _div_stats(acc, b3, 8, mean_heads=True)
    return _pool(out0, s1, s2, batch32, gn3w, gn3b, gn3m, Wlin, blin)


# speedup vs baseline: 106.6805x; 1.0132x over previous
import functools

import jax
import jax.numpy as jnp
from jax import lax
from jax.experimental import pallas as pl
from jax.experimental.pallas import tpu as pltpu
from jax.experimental.pallas import tpu_sc as plsc

N = 10000
E = 320000
DIN = 128
H = 8
NG = 16

NE = E + N
NCORE = 2
NSUB = 16
EPT = 20736
E_PAD = EPT * NSUB
NPAD = 10112
ROWS_PT = NPAD // NSUB
HH = H // NCORE


def _sc_edge_kernel(ch, EB, unroll=2):
    F2 = HH * ch
    OUTW = F2 + 16
    NV = F2 // 16
    NBLK = EPT // EB

    mesh = plsc.VectorSubcoreMesh(core_axis_name="c", subcore_axis_name="s")

    idx_t = pltpu.VMEM((EB,), jnp.int32)
    row_t = pltpu.VMEM((EB, F2), jnp.float32)
    stage_t = pltpu.VMEM((EB, OUTW), jnp.float32)

    @functools.partial(
        pl.kernel,
        out_type=jax.ShapeDtypeStruct((NCORE, NPAD, OUTW), jnp.float32),
        mesh=mesh,
        scratch_types=(
            [idx_t] * 8
            + [row_t] * 4
            + [stage_t] * 2
            + [pltpu.VMEM((F2,), jnp.float32)]
            + [pltpu.VMEM_SHARED((NPAD, OUTW), jnp.float32)]
            + [pltpu.SemaphoreType.DMA] * 8
        ),
        compiler_params=pltpu.CompilerParams(use_tc_tiling_on_sc=False,
                                             needs_layout_passes=False),
    )
    def kern(xls, xrs, sadj_hbm, dadj_hbm, draw_hbm, att_hbm, out_hbm,
             sadjA, dadjA, drawA, dscatA, sadjB, dadjB, drawB, dscatB,
             xlA, xrA, xlB, xrB, stageA, stageB, att_v, acc,
             sem_iA, sem_iB, sem_xlA, sem_xlB, sem_xrA, sem_xrB,
             sem_scA, sem_scB):
        c = lax.axis_index("c")
        s = lax.axis_index("s")
        zeros16 = jnp.zeros((16,), jnp.float32)

        A = (sadjA, dadjA, drawA, dscatA, xlA, xrA, stageA,
             sem_iA, sem_xlA, sem_xrA, sem_scA)
        B = (sadjB, dadjB, drawB, dscatB, xlB, xrB, stageB,
             sem_iB, sem_xlB, sem_xrB, sem_scB)

        def zrow(r, _):
            for i in range(OUTW // 16):
                stageA[r, pl.ds(16 * i, 16)] = zeros16
            return 0
        lax.fori_loop(0, EB, zrow, 0)

        r0 = s * ROWS_PT
        for off in range(0, ROWS_PT, EB):
            nrows = min(EB, ROWS_PT - off)
            pltpu.sync_copy(stageA.at[pl.ds(0, nrows)],
                            acc.at[pl.ds(r0 + off, nrows)])
        plsc.subcore_barrier()

        pltpu.sync_copy(att_hbm.at[c], att_v)
        attv = [att_v[pl.ds(16 * i, 16)] for i in range(NV)]
        lane = lax.iota(jnp.int32, 16)
        ebase = s * EPT

        def idx_srcs(blk):
            base = ebase + blk * EB
            return (sadj_hbm.at[c, pl.ds(base, EB)],
                    dadj_hbm.at[c, pl.ds(base, EB)],
                    draw_hbm.at[pl.ds(base, EB)])

        def issue_idx(st, blk):
            sa, da, dr = idx_srcs(blk)
            pltpu.async_copy(sa, st[0], st[7])
            pltpu.async_copy(da, st[1], st[7])
            pltpu.async_copy(dr, st[2], st[7])

        def wait_idx(st, blk):
            sa, da, dr = idx_srcs(blk)
            pltpu.make_async_copy(sa, st[0], st[7]).wait()
            pltpu.make_async_copy(da, st[1], st[7]).wait()
            pltpu.make_async_copy(dr, st[2], st[7]).wait()

        def issue_gathers(st):
            pltpu.async_copy(xls.at[st[0]], st[4], st[8])
            pltpu.async_copy(xrs.at[st[1]], st[5], st[9])

        def wait_gathers(st):
            pltpu.make_async_copy(xls.at[st[0]], st[4], st[8]).wait()
            pltpu.make_async_copy(xrs.at[st[1]], st[5], st[9]).wait()

        def issue_scatter(st):
            pltpu.async_copy(st[6], acc.at[st[3]], st[10], add=True)

        def wait_scatter(st):
            pltpu.make_async_copy(st[6], acc.at[st[3]], st[10]).wait()

        def snap_scatter_idx(st):
            for i in range(EB // 16):
                sl = pl.ds(16 * i, 16)
                st[3][sl] = st[2][sl]

        def compute(st):
            xl_buf, xr_buf, stage = st[4], st[5], st[6]

            def edge_body(e):
                xlv = []
                lv = []
                for i in range(NV):
                    sl = pl.ds(16 * i, 16)
                    xv = xl_buf[e, sl]
                    z = xv + xr_buf[e, sl]
                    xlv.append(xv)
                    lv.append(jnp.maximum(z, 0.2 * z))
                ws = []
                whs = []
                if ch >= 16:
                    CV = ch // 16
                    for h in range(HH):
                        t = lv[h * CV] * attv[h * CV]
                        for j in range(1, CV):
                            t = t + lv[h * CV + j] * attv[h * CV + j]
                        sh = jnp.sum(t)
                        wh = jnp.exp(jnp.broadcast_to(sh, (16,)))
                        ws.append(wh)
                        whs.append(wh)
                else:
                    m8 = lane < 8
                    for v in range(NV):
                        t = lv[v] * attv[v]
                        tot = jnp.sum(t)
                        first = jnp.sum(jnp.where(m8, t, 0.0))
                        we = jnp.exp(jnp.broadcast_to(first, (16,)))
                        wo = jnp.exp(jnp.broadcast_to(tot - first, (16,)))
                        ws.append(jnp.where(m8, we, wo))
                        whs.append(we)
                        whs.append(wo)
                if ch >= 16:
                    CV = ch // 16
                    for i in range(NV):
                        stage[e, pl.ds(16 * i, 16)] = xlv[i] * ws[i // CV]
                else:
                    for i in range(NV):
                        stage[e, pl.ds(16 * i, 16)] = xlv[i] * ws[i]
                denv = jnp.zeros((16,), jnp.float32)
                for h in range(HH):
                    denv = jnp.where(lane == h, whs[h], denv)
                stage[e, pl.ds(F2, 16)] = denv

            plsc.parallel_loop(0, EB, 1, unroll=unroll)(edge_body)

        for ref, src in zip((sadjA, dadjA, drawA), idx_srcs(0)):
            pltpu.sync_copy(src, ref)
        issue_gathers(A)
        issue_idx(B, 1)

        def pair_body(b2, _):
            blk0 = 2 * b2
            wait_gathers(A)
            wait_idx(B, blk0 + 1)
            issue_gathers(B)

            @pl.when(b2 > 0)
            def _():
                wait_scatter(A)
            snap_scatter_idx(A)

            @pl.when(blk0 + 2 < NBLK)
            def _():
                issue_idx(A, blk0 + 2)
            compute(A)
            issue_scatter(A)

            wait_gathers(B)

            @pl.when(blk0 + 2 < NBLK)
            def _():
                wait_idx(A, blk0 + 2)
                issue_gathers(A)

            @pl.when(b2 > 0)
            def _():
                wait_scatter(B)
            snap_scatter_idx(B)

            @pl.when(blk0 + 3 < NBLK)
            def _():
                issue_idx(B, blk0 + 3)
            compute(B)
            issue_scatter(B)
            return 0

        lax.fori_loop(0, NBLK // 2, pair_body, 0)
        wait_scatter(A)
        wait_scatter(B)

        plsc.subcore_barrier()
        for off in range(0, ROWS_PT, EB):
            nrows = min(EB, ROWS_PT - off)
            pltpu.sync_copy(acc.at[pl.ds(r0 + off, nrows)],
                            out_hbm.at[c, pl.ds(r0 + off, nrows)])

    return kern


def _mm_pair(x, wl, wr, F2):
    K = x.shape[1]
    TR = 2000
    R = N // TR
    wlh = wl.reshape(K, NCORE, F2).transpose(1, 0, 2)
    wrh = wr.reshape(K, NCORE, F2).transpose(1, 0, 2)

    def mm_kernel(x_ref, wl_ref, wr_ref, xl_out, xr_out):
        xb = x_ref[...]
        xl_out[0] = jnp.dot(xb, wl_ref[0], preferred_element_type=jnp.float32)
        xr_out[0] = jnp.dot(xb, wr_ref[0], preferred_element_type=jnp.float32)

    out_sh = jax.ShapeDtypeStruct((NCORE, N, F2), jnp.float32)
    xl3, xr3 = pl.pallas_call(
        mm_kernel,
        grid=(NCORE, R),
        in_specs=[
            pl.BlockSpec((TR, K), lambda c, r: (r, 0)),
            pl.BlockSpec((1, K, F2), lambda c, r: (c, 0, 0)),
            pl.BlockSpec((1, K, F2), lambda c, r: (c, 0, 0)),
        ],
        out_specs=[
            pl.BlockSpec((1, TR, F2), lambda c, r: (c, r, 0)),
            pl.BlockSpec((1, TR, F2), lambda c, r: (c, r, 0)),
        ],
        out_shape=[out_sh, out_sh],
    )(x, wlh, wrh)
    return xl3.reshape(NCORE * N, F2), xr3.reshape(NCORE * N, F2)


TRW = 1264
RSTEPS = NPAD // TRW


def _div_stats(acc, bias, ch, mean_heads):
    F2 = HH * ch
    F = H * ch
    Fout = ch if mean_heads else F
    OUTW = F2 + 16

    def k(acc_ref, b_ref, out_ref, s1_ref, s2_ref):
        r = pl.program_id(0)
        parts = []
        for c in range(NCORE):
            num = acc_ref[c, :, :F2]
            for h in range(HH):
                den = acc_ref[c, :, F2 + h:F2 + h + 1]
                parts.append(num[:, h * ch:(h + 1) * ch] / (den + 1e-16))
        if mean_heads:
            t = parts[0]
            for p in parts[1:]:
                t = t + p
            out0 = t / float(H) + b_ref[...]
        else:
            out0 = jnp.concatenate(parts, axis=1) + b_ref[...]
        out_ref[...] = out0
        row = r * TRW + lax.broadcasted_iota(jnp.int32, (TRW, 1), 0)
        mask = row < N
        c1 = jnp.sum(jnp.where(mask, out0, 0.0), axis=0, keepdims=True)
        c2 = jnp.sum(jnp.where(mask, out0 * out0, 0.0), axis=0,
                     keepdims=True)

        @pl.when(r == 0)
        def _():
            s1_ref[...] = c1
            s2_ref[...] = c2

        @pl.when(r > 0)
        def _():
            s1_ref[...] += c1
            s2_ref[...] += c2

    stat_sh = jax.ShapeDtypeStruct((1, Fout), jnp.float32)
    return pl.pallas_call(
        k,
        grid=(RSTEPS,),
        in_specs=[
            pl.BlockSpec((NCORE, TRW, OUTW), lambda r: (0, r, 0)),
            pl.BlockSpec((1, Fout), lambda r: (0, 0)),
        ],
        out_specs=[
            pl.BlockSpec((TRW, Fout), lambda r: (r, 0)),
            pl.BlockSpec((1, Fout), lambda r: (0, 0)),
            pl.BlockSpec((1, Fout), lambda r: (0, 0)),
        ],
        out_shape=[jax.ShapeDtypeStruct((N, Fout), jnp.float32),
                   stat_sh, stat_sh],
    )(acc, bias.reshape(1, Fout))


def _norm_mm_pair(out0, s1, s2, gw, gb, gm, wl, wr, F2):
    F = out0.shape[1]
    TR = 2000
    R = N // TR
    wlh = wl.reshape(F, NCORE, F2).transpose(1, 0, 2)
    wrh = wr.reshape(F, NCORE, F2).transpose(1, 0, 2)

    def k(x_ref, s1_ref, s2_ref, gw_ref, gb_ref, gm_ref, wl_ref, wr_ref,
          xl_out, xr_out):
        mean = s1_ref[...] * (1.0 / N)
        ex2 = s2_ref[...] * (1.0 / N)
        gm_ = gm_ref[...]
        var = ex2 + (gm_ * gm_ - 2.0 * gm_) * mean * mean
        inv = gw_ref[...] / jnp.sqrt(var + 1e-5)
        hb = jnp.maximum((x_ref[...] - gm_ * mean) * inv + gb_ref[...], 0.0)
        xl_out[0] = jnp.dot(hb, wl_ref[0], preferred_element_type=jnp.float32)
        xr_out[0] = jnp.dot(hb, wr_ref[0], preferred_element_type=jnp.float32)

    vec = pl.BlockSpec((1, F), lambda c, r: (0, 0))
    out_sh = jax.ShapeDtypeStruct((NCORE, N, F2), jnp.float32)
    xl3, xr3 = pl.pallas_call(
        k,
        grid=(NCORE, R),
        in_specs=[pl.BlockSpec((TR, F), lambda c, r: (r, 0)),
                  vec, vec, vec, vec, vec,
                  pl.BlockSpec((1, F, F2), lambda c, r: (c, 0, 0)),
                  pl.BlockSpec((1, F, F2), lambda c, r: (c, 0, 0))],
        out_specs=[
            pl.BlockSpec((1, TR, F2), lambda c, r: (c, r, 0)),
            pl.BlockSpec((1, TR, F2), lambda c, r: (c, r, 0)),
        ],
        out_shape=[out_sh, out_sh],
    )(out0, s1, s2, gw.reshape(1, F), gb.reshape(1, F), gm.reshape(1, F),
      wlh, wrh)
    return xl3.reshape(NCORE * N, F2), xr3.reshape(NCORE * N, F2)


def _pool(out0, s1, s2, batch, gw, gb, gm, wlin, blin):
    def k(x_ref, s1_ref, s2_ref, batch_ref, gw_ref, gb_ref, gm_ref,
          wlin_ref, blin_ref, logits_ref, pooled_ref):
        mean = s1_ref[...] * (1.0 / N)
        ex2 = s2_ref[...] * (1.0 / N)
        gm_ = gm_ref[...]
        var = ex2 + (gm_ * gm_ - 2.0 * gm_) * mean * mean
        inv = gw_ref[...] / jnp.sqrt(var + 1e-5)
        h3 = jnp.maximum((x_ref[...] - gm_ * mean) * inv + gb_ref[...], 0.0)
        gid = lax.broadcasted_iota(jnp.int32, (1, NG), 1)
        oh = (batch_ref[...] == gid).astype(jnp.float32)
        psum = lax.dot_general(oh, h3, (((0,), (0,)), ((), ())),
                               preferred_element_type=jnp.float32)
        counts = jnp.sum(oh, axis=0, keepdims=True)
        pooled = psum / jnp.maximum(counts, 1.0).reshape(NG, 1)
        pooled_ref[...] = pooled
        logits_ref[...] = jnp.dot(pooled, wlin_ref[...],
                                  preferred_element_type=jnp.float32) \
            + blin_ref[...]

    return pl.pallas_call(
        k,
        out_shape=[jax.ShapeDtypeStruct((NG, 4), jnp.float32),
                   jax.ShapeDtypeStruct((NG, 8), jnp.float32)],
    )(out0, s1, s2, batch.reshape(N, 1), gw.reshape(1, 8), gb.reshape(1, 8),
      gm.reshape(1, 8), wlin, blin.reshape(1, 4))


_sc_l1 = _sc_edge_kernel(32, 48)
_sc_l2 = _sc_edge_kernel(16, 128)
_sc_l3 = _sc_edge_kernel(8, 128)


def kernel(x, edge_index, batch, W1l, W1r, a1, b1, gn1w, gn1b, gn1m,
           W2l, W2r, a2, b2, gn2w, gn2b, gn2m,
           W3l, W3r, a3, b3, gn3w, gn3b, gn3m, Wlin, blin):
    src = jnp.concatenate([
        edge_index[0].astype(jnp.int32),
        jnp.arange(N, dtype=jnp.int32),
        jnp.zeros((E_PAD - NE,), jnp.int32),
    ])
    dst = jnp.concatenate([
        edge_index[1].astype(jnp.int32),
        jnp.arange(N, dtype=jnp.int32),
        jnp.full((E_PAD - NE,), N, jnp.int32),
    ])
    coff = jnp.array([[0], [N]], jnp.int32)
    sadj = src[None, :] + coff
    dadj = dst[None, :] + coff
    batch32 = batch.astype(jnp.int32)

    xls, xrs = _mm_pair(x, W1l, W1r, HH * 32)
    acc = _sc_l1(xls, xrs, sadj, dadj, dst, a1.reshape(NCORE, HH * 32))
    out0, s1, s2 = _div_stats(acc, b1, 32, mean_heads=False)
    xls, xrs = _norm_mm_pair(out0, s1, s2, gn1w, gn1b, gn1m, W2l, W2r,
                             HH * 16)
    acc = _sc_l2(xls, xrs, sadj, dadj, dst, a2.reshape(NCORE, HH * 16))
    out0, s1, s2 = _div_stats(acc, b2, 16, mean_heads=False)
    xls, xrs = _norm_mm_pair(out0, s1, s2, gn2w, gn2b, gn2m, W3l, W3r,
                             HH * 8)
    acc = _sc_l3(xls, xrs, sadj, dadj, dst, a3.reshape(NCORE, HH * 8))
    out0, s1, s2 = _div_stats(acc, b3, 8, mean_heads=True)
    return _pool(out0, s1, s2, batch32, gn3w, gn3b, gn3m, Wlin, blin)


# speedup vs baseline: 124.3484x; 1.1656x over previous
import functools

import numpy as np

import jax
import jax.numpy as jnp
from jax import lax
from jax.experimental import pallas as pl
from jax.experimental.pallas import tpu as pltpu
from jax.experimental.pallas import tpu_sc as plsc

N = 10000
E = 320000
DIN = 128
H = 8
NG = 16

NE = E + N
NCORE = 2
NSUB = 16
EPT = 20736
E_PAD = EPT * NSUB
NPAD = 10112
ROWS_PT = NPAD // NSUB
HH = H // NCORE


def _pair_perm(F2):
    perm = np.empty((F2,), np.int32)
    for i in range(F2 // 32):
        for t in range(16):
            perm[32 * i + 2 * t] = 32 * i + t
            perm[32 * i + 2 * t + 1] = 32 * i + 16 + t
    return perm


def _sc_edge_kernel(ch, EB, unroll=2):
    F2 = HH * ch
    OUTW = F2 + 16
    NV = F2 // 16
    NBLK = EPT // EB

    mesh = plsc.VectorSubcoreMesh(core_axis_name="c", subcore_axis_name="s")

    idx_t = pltpu.VMEM((EB,), jnp.int32)
    row_t = pltpu.VMEM((EB, F2), jnp.bfloat16)
    stage_t = pltpu.VMEM((EB, OUTW), jnp.float32)

    @functools.partial(
        pl.kernel,
        out_type=jax.ShapeDtypeStruct((NCORE, NPAD, OUTW), jnp.float32),
        mesh=mesh,
        scratch_types=(
            [idx_t] * 8
            + [row_t] * 4
            + [stage_t] * 2
            + [pltpu.VMEM((F2,), jnp.float32)]
            + [pltpu.VMEM_SHARED((NPAD, OUTW), jnp.float32)]
            + [pltpu.SemaphoreType.DMA] * 8
        ),
        compiler_params=pltpu.CompilerParams(use_tc_tiling_on_sc=False,
                                             needs_layout_passes=False),
    )
    def kern(xls, xrs, sadj_hbm, dadj_hbm, draw_hbm, att_hbm, out_hbm,
             sadjA, dadjA, drawA, dscatA, sadjB, dadjB, drawB, dscatB,
             xlA, xrA, xlB, xrB, stageA, stageB, att_v, acc,
             sem_iA, sem_iB, sem_xlA, sem_xlB, sem_xrA, sem_xrB,
             sem_scA, sem_scB):
        c = lax.axis_index("c")
        s = lax.axis_index("s")
        zeros16 = jnp.zeros((16,), jnp.float32)

        A = (sadjA, dadjA, drawA, dscatA, xlA, xrA, stageA,
             sem_iA, sem_xlA, sem_xrA, sem_scA)
        B = (sadjB, dadjB, drawB, dscatB, xlB, xrB, stageB,
             sem_iB, sem_xlB, sem_xrB, sem_scB)

        def zrow(r, _):
            for i in range(OUTW // 16):
                stageA[r, pl.ds(16 * i, 16)] = zeros16
            return 0
        lax.fori_loop(0, EB, zrow, 0)

        r0 = s * ROWS_PT
        for off in range(0, ROWS_PT, EB):
            nrows = min(EB, ROWS_PT - off)
            pltpu.sync_copy(stageA.at[pl.ds(0, nrows)],
                            acc.at[pl.ds(r0 + off, nrows)])
        plsc.subcore_barrier()

        pltpu.sync_copy(att_hbm.at[c], att_v)
        attv = [att_v[pl.ds(16 * i, 16)] for i in range(NV)]
        lane = lax.iota(jnp.int32, 16)
        ebase = s * EPT

        def idx_srcs(blk):
            base = ebase + blk * EB
            return (sadj_hbm.at[c, pl.ds(base, EB)],
                    dadj_hbm.at[c, pl.ds(base, EB)],
                    draw_hbm.at[pl.ds(base, EB)])

        def issue_idx(st, blk):
            sa, da, dr = idx_srcs(blk)
            pltpu.async_copy(sa, st[0], st[7])
            pltpu.async_copy(da, st[1], st[7])
            pltpu.async_copy(dr, st[2], st[7])

        def wait_idx(st, blk):
            sa, da, dr = idx_srcs(blk)
            pltpu.make_async_copy(sa, st[0], st[7]).wait()
            pltpu.make_async_copy(da, st[1], st[7]).wait()
            pltpu.make_async_copy(dr, st[2], st[7]).wait()

        def issue_gathers(st):
            pltpu.async_copy(xls.at[st[0]], st[4], st[8])
            pltpu.async_copy(xrs.at[st[1]], st[5], st[9])

        def wait_gathers(st):
            pltpu.make_async_copy(xls.at[st[0]], st[4], st[8]).wait()
            pltpu.make_async_copy(xrs.at[st[1]], st[5], st[9]).wait()

        def issue_scatter(st):
            pltpu.async_copy(st[6], acc.at[st[3]], st[10], add=True)

        def wait_scatter(st):
            pltpu.make_async_copy(st[6], acc.at[st[3]], st[10]).wait()

        def snap_scatter_idx(st):
            for i in range(EB // 16):
                sl = pl.ds(16 * i, 16)
                st[3][sl] = st[2][sl]

        def compute(st):
            xl_buf, xr_buf, stage = st[4], st[5], st[6]

            def edge_body(e):
                xlv = []
                lv = []
                for p in range(NV // 2):
                    sl = pl.ds(32 * p, 32)
                    xa, xb = plsc.unpack(
                        xl_buf[e, sl], format=plsc.PackFormat.INTERLEAVED)
                    ra, rb = plsc.unpack(
                        xr_buf[e, sl], format=plsc.PackFormat.INTERLEAVED)
                    for xv, rv in ((xa, ra), (xb, rb)):
                        z = xv + rv
                        xlv.append(xv)
                        lv.append(jnp.maximum(z, 0.2 * z))
                ws = []
                whs = []
                if ch >= 16:
                    CV = ch // 16
                    for h in range(HH):
                        t = lv[h * CV] * attv[h * CV]
                        for j in range(1, CV):
                            t = t + lv[h * CV + j] * attv[h * CV + j]
                        sh = jnp.sum(t)
                        wh = jnp.exp(jnp.broadcast_to(sh, (16,)))
                        ws.append(wh)
                        whs.append(wh)
                else:
                    m8 = lane < 8
                    for v in range(NV):
                        t = lv[v] * attv[v]
                        tot = jnp.sum(t)
                        first = jnp.sum(jnp.where(m8, t, 0.0))
                        we = jnp.exp(jnp.broadcast_to(first, (16,)))
                        wo = jnp.exp(jnp.broadcast_to(tot - first, (16,)))
                        ws.append(jnp.where(m8, we, wo))
                        whs.append(we)
                        whs.append(wo)
                if ch >= 16:
                    CV = ch // 16
                    for i in range(NV):
                        stage[e, pl.ds(16 * i, 16)] = xlv[i] * ws[i // CV]
                else:
                    for i in range(NV):
                        stage[e, pl.ds(16 * i, 16)] = xlv[i] * ws[i]
                denv = jnp.zeros((16,), jnp.float32)
                for h in range(HH):
                    denv = jnp.where(lane == h, whs[h], denv)
                stage[e, pl.ds(F2, 16)] = denv

            plsc.parallel_loop(0, EB, 1, unroll=unroll)(edge_body)

        for ref, src in zip((sadjA, dadjA, drawA), idx_srcs(0)):
            pltpu.sync_copy(src, ref)
        issue_gathers(A)
        issue_idx(B, 1)

        def pair_body(b2, _):
            blk0 = 2 * b2
            wait_gathers(A)
            wait_idx(B, blk0 + 1)
            issue_gathers(B)

            @pl.when(b2 > 0)
            def _():
                wait_scatter(A)
            snap_scatter_idx(A)

            @pl.when(blk0 + 2 < NBLK)
            def _():
                issue_idx(A, blk0 + 2)
            compute(A)
            issue_scatter(A)

            wait_gathers(B)

            @pl.when(blk0 + 2 < NBLK)
            def _():
                wait_idx(A, blk0 + 2)
                issue_gathers(A)

            @pl.when(b2 > 0)
            def _():
                wait_scatter(B)
            snap_scatter_idx(B)

            @pl.when(blk0 + 3 < NBLK)
            def _():
                issue_idx(B, blk0 + 3)
            compute(B)
            issue_scatter(B)
            return 0

        lax.fori_loop(0, NBLK // 2, pair_body, 0)
        wait_scatter(A)
        wait_scatter(B)

        plsc.subcore_barrier()
        for off in range(0, ROWS_PT, EB):
            nrows = min(EB, ROWS_PT - off)
            pltpu.sync_copy(acc.at[pl.ds(r0 + off, nrows)],
                            out_hbm.at[c, pl.ds(r0 + off, nrows)])

    return kern


def _mm_pair(x, wl, wr, F2):
    K = x.shape[1]
    TR = 2000
    R = N // TR
    perm = _pair_perm(F2)
    wlh = wl.reshape(K, NCORE, F2).transpose(1, 0, 2)[:, :, perm]
    wrh = wr.reshape(K, NCORE, F2).transpose(1, 0, 2)[:, :, perm]

    def mm_kernel(x_ref, wl_ref, wr_ref, xl_out, xr_out):
        xb = x_ref[...]
        xl_out[0] = jnp.dot(
            xb, wl_ref[0],
            preferred_element_type=jnp.float32).astype(jnp.bfloat16)
        xr_out[0] = jnp.dot(
            xb, wr_ref[0],
            preferred_element_type=jnp.float32).astype(jnp.bfloat16)

    out_sh = jax.ShapeDtypeStruct((NCORE, N, F2), jnp.bfloat16)
    xl3, xr3 = pl.pallas_call(
        mm_kernel,
        grid=(NCORE, R),
        in_specs=[
            pl.BlockSpec((TR, K), lambda c, r: (r, 0)),
            pl.BlockSpec((1, K, F2), lambda c, r: (c, 0, 0)),
            pl.BlockSpec((1, K, F2), lambda c, r: (c, 0, 0)),
        ],
        out_specs=[
            pl.BlockSpec((1, TR, F2), lambda c, r: (c, r, 0)),
            pl.BlockSpec((1, TR, F2), lambda c, r: (c, r, 0)),
        ],
        out_shape=[out_sh, out_sh],
    )(x, wlh, wrh)
    return xl3.reshape(NCORE * N, F2), xr3.reshape(NCORE * N, F2)


TRW = 1264
RSTEPS = NPAD // TRW


def _div_stats(acc, bias, ch, mean_heads):
    F2 = HH * ch
    F = H * ch
    Fout = ch if mean_heads else F
    OUTW = F2 + 16

    def k(acc_ref, b_ref, out_ref, s1_ref, s2_ref):
        r = pl.program_id(0)
        parts = []
        for c in range(NCORE):
            num = acc_ref[c, :, :F2]
            for h in range(HH):
                den = acc_ref[c, :, F2 + h:F2 + h + 1]
                parts.append(num[:, h * ch:(h + 1) * ch] / (den + 1e-16))
        if mean_heads:
            t = parts[0]
            for p in parts[1:]:
                t = t + p
            out0 = t / float(H) + b_ref[...]
        else:
            out0 = jnp.concatenate(parts, axis=1) + b_ref[...]
        out_ref[...] = out0
        row = r * TRW + lax.broadcasted_iota(jnp.int32, (TRW, 1), 0)
        mask = row < N
        c1 = jnp.sum(jnp.where(mask, out0, 0.0), axis=0, keepdims=True)
        c2 = jnp.sum(jnp.where(mask, out0 * out0, 0.0), axis=0,
                     keepdims=True)

        @pl.when(r == 0)
        def _():
            s1_ref[...] = c1
            s2_ref[...] = c2

        @pl.when(r > 0)
        def _():
            s1_ref[...] += c1
            s2_ref[...] += c2

    stat_sh = jax.ShapeDtypeStruct((1, Fout), jnp.float32)
    return pl.pallas_call(
        k,
        grid=(RSTEPS,),
        in_specs=[
            pl.BlockSpec((NCORE, TRW, OUTW), lambda r: (0, r, 0)),
            pl.BlockSpec((1, Fout), lambda r: (0, 0)),
        ],
        out_specs=[
            pl.BlockSpec((TRW, Fout), lambda r: (r, 0)),
            pl.BlockSpec((1, Fout), lambda r: (0, 0)),
            pl.BlockSpec((1, Fout), lambda r: (0, 0)),
        ],
        out_shape=[jax.ShapeDtypeStruct((N, Fout), jnp.float32),
                   stat_sh, stat_sh],
    )(acc, bias.reshape(1, Fout))


def _norm_mm_pair(out0, s1, s2, gw, gb, gm, wl, wr, F2):
    F = out0.shape[1]
    TR = 2000
    R = N // TR
    perm = _pair_perm(F2)
    wlh = wl.reshape(F, NCORE, F2).transpose(1, 0, 2)[:, :, perm]
    wrh = wr.reshape(F, NCORE, F2).transpose(1, 0, 2)[:, :, perm]

    def k(x_ref, s1_ref, s2_ref, gw_ref, gb_ref, gm_ref, wl_ref, wr_ref,
          xl_out, xr_out):
        mean = s1_ref[...] * (1.0 / N)
        ex2 = s2_ref[...] * (1.0 / N)
        gm_ = gm_ref[...]
        var = ex2 + (gm_ * gm_ - 2.0 * gm_) * mean * mean
        inv = gw_ref[...] / jnp.sqrt(var + 1e-5)
        hb = jnp.maximum((x_ref[...] - gm_ * mean) * inv + gb_ref[...], 0.0)
        xl_out[0] = jnp.dot(
            hb, wl_ref[0],
            preferred_element_type=jnp.float32).astype(jnp.bfloat16)
        xr_out[0] = jnp.dot(
            hb, wr_ref[0],
            preferred_element_type=jnp.float32).astype(jnp.bfloat16)

    vec = pl.BlockSpec((1, F), lambda c, r: (0, 0))
    out_sh = jax.ShapeDtypeStruct((NCORE, N, F2), jnp.bfloat16)
    xl3, xr3 = pl.pallas_call(
        k,
        grid=(NCORE, R),
        in_specs=[pl.BlockSpec((TR, F), lambda c, r: (r, 0)),
                  vec, vec, vec, vec, vec,
                  pl.BlockSpec((1, F, F2), lambda c, r: (c, 0, 0)),
                  pl.BlockSpec((1, F, F2), lambda c, r: (c, 0, 0))],
        out_specs=[
            pl.BlockSpec((1, TR, F2), lambda c, r: (c, r, 0)),
            pl.BlockSpec((1, TR, F2), lambda c, r: (c, r, 0)),
        ],
        out_shape=[out_sh, out_sh],
    )(out0, s1, s2, gw.reshape(1, F), gb.reshape(1, F), gm.reshape(1, F),
      wlh, wrh)
    return xl3.reshape(NCORE * N, F2), xr3.reshape(NCORE * N, F2)


def _pool(out0, s1, s2, batch, gw, gb, gm, wlin, blin):
    def k(x_ref, s1_ref, s2_ref, batch_ref, gw_ref, gb_ref, gm_ref,
          wlin_ref, blin_ref, logits_ref, pooled_ref):
        mean = s1_ref[...] * (1.0 / N)
        ex2 = s2_ref[...] * (1.0 / N)
        gm_ = gm_ref[...]
        var = ex2 + (gm_ * gm_ - 2.0 * gm_) * mean * mean
        inv = gw_ref[...] / jnp.sqrt(var + 1e-5)
        h3 = jnp.maximum((x_ref[...] - gm_ * mean) * inv + gb_ref[...], 0.0)
        gid = lax.broadcasted_iota(jnp.int32, (1, NG), 1)
        oh = (batch_ref[...] == gid).astype(jnp.float32)
        psum = lax.dot_general(oh, h3, (((0,), (0,)), ((), ())),
                               preferred_element_type=jnp.float32)
        counts = jnp.sum(oh, axis=0, keepdims=True)
        pooled = psum / jnp.maximum(counts, 1.0).reshape(NG, 1)
        pooled_ref[...] = pooled
        logits_ref[...] = jnp.dot(pooled, wlin_ref[...],
                                  preferred_element_type=jnp.float32) \
            + blin_ref[...]

    return pl.pallas_call(
        k,
        out_shape=[jax.ShapeDtypeStruct((NG, 4), jnp.float32),
                   jax.ShapeDtypeStruct((NG, 8), jnp.float32)],
    )(out0, s1, s2, batch.reshape(N, 1), gw.reshape(1, 8), gb.reshape(1, 8),
      gm.reshape(1, 8), wlin, blin.reshape(1, 4))


_sc_l1 = _sc_edge_kernel(32, 64)
_sc_l2 = _sc_edge_kernel(16, 128)
_sc_l3 = _sc_edge_kernel(8, 128)


def kernel(x, edge_index, batch, W1l, W1r, a1, b1, gn1w, gn1b, gn1m,
           W2l, W2r, a2, b2, gn2w, gn2b, gn2m,
           W3l, W3r, a3, b3, gn3w, gn3b, gn3m, Wlin, blin):
    src = jnp.concatenate([
        edge_index[0].astype(jnp.int32),
        jnp.arange(N, dtype=jnp.int32),
        jnp.zeros((E_PAD - NE,), jnp.int32),
    ])
    dst = jnp.concatenate([
        edge_index[1].astype(jnp.int32),
        jnp.arange(N, dtype=jnp.int32),
        jnp.full((E_PAD - NE,), N, jnp.int32),
    ])
    coff = jnp.array([[0], [N]], jnp.int32)
    sadj = src[None, :] + coff
    dadj = dst[None, :] + coff
    batch32 = batch.astype(jnp.int32)

    xls, xrs = _mm_pair(x, W1l, W1r, HH * 32)
    acc = _sc_l1(xls, xrs, sadj, dadj, dst, a1.reshape(NCORE, HH * 32))
    out0, s1, s2 = _div_stats(acc, b1, 32, mean_heads=False)
    xls, xrs = _norm_mm_pair(out0, s1, s2, gn1w, gn1b, gn1m, W2l, W2r,
                             HH * 16)
    acc = _sc_l2(xls, xrs, sadj, dadj, dst, a2.reshape(NCORE, HH * 16))
    out0, s1, s2 = _div_stats(acc, b2, 16, mean_heads=False)
    xls, xrs = _norm_mm_pair(out0, s1, s2, gn2w, gn2b, gn2m, W3l, W3r,
                             HH * 8)
    acc = _sc_l3(xls, xrs, sadj, dadj, dst, a3.reshape(NCORE, HH * 8))
    out0, s1, s2 = _div_stats(acc, b3, 8, mean_heads=True)
    return _pool(out0, s1, s2, batch32, gn3w, gn3b, gn3m, Wlin, blin)
